# baseline (speedup 1.0000x reference)
import functools

import jax
import jax.numpy as jnp
from jax import lax
from jax.experimental import pallas as pl
from jax.experimental.pallas import tpu as pltpu
from jax.experimental.pallas import tpu_sc as plsc

N = 32768
D = 64
K = 1024

NCHUNKS = 4
NT = N // NCHUNKS

BN = 512
NB = NT // BN

NC = 2
NS = 16
NW = NC * NS
B_PER_W = NT // NW
CHUNK = 128
NCHUNK = B_PER_W // CHUNK
W_PER_ROW = BN // B_PER_W
DP = 128


def _argmin_body(zt_ref, cbt_ref, out_ref):
    zt = zt_ref[...]
    zsq = jnp.sum(zt * zt, axis=0, keepdims=True)
    cbt = cbt_ref[...]
    ones = jnp.ones((D, 1), jnp.float32)
    cbsq = lax.dot_general(
        cbt * cbt, ones, (((0,), (0,)), ((), ())),
        preferred_element_type=jnp.float32)
    mmneg2 = lax.dot_general(
        cbt * -2.0, zt, (((0,), (0,)), ((), ())),
        preferred_element_type=jnp.float32)
    zsq8 = jnp.broadcast_to(zsq, (8, BN))
    iota8 = lax.broadcasted_iota(jnp.int32, (8, BN), 0)
    rm = jnp.full((8, BN), jnp.inf, jnp.float32)
    ri = jnp.full((8, BN), K, jnp.int32)
    for j in range(K // 8):
        mm_j = lax.slice(mmneg2, (j * 8, 0), (j * 8 + 8, BN))
        cb_j = lax.slice(cbsq, (j * 8, 0), (j * 8 + 8, 1))
        d = (zsq8 + mm_j) + cb_j
        lt = d < rm
        rm = jnp.where(lt, d, rm)
        ri = jnp.where(lt, iota8 + (j * 8), ri)
    mn = jnp.min(rm, axis=0, keepdims=True)
    idx = jnp.min(jnp.where(rm == mn, ri, K), axis=0, keepdims=True)
    out_ref[pl.ds(pl.program_id(0), 1), :] = idx


def _make_argmin_call(c):
    return pl.pallas_call(
        _argmin_body,
        grid=(NB,),
        in_specs=[
            pl.BlockSpec((D, BN), lambda i, c=c: (0, c * NB + i)),
            pl.BlockSpec((D, K), lambda i: (0, 0)),
        ],
        out_specs=pl.BlockSpec((NB, BN), lambda i: (0, 0)),
        out_shape=jax.ShapeDtypeStruct((NB, BN), jnp.int32),
    )


_argmin_calls = [_make_argmin_call(c) for c in range(NCHUNKS)]


def _gather_body(table_hbm, idx_hbm, out_hbm, idx_v, rows_v, sem):
    wid = lax.axis_index("s") * NC + lax.axis_index("c")
    row = wid // W_PER_ROW
    col = (wid % W_PER_ROW) * B_PER_W
    pltpu.sync_copy(
        idx_hbm.at[pl.ds(row, 1), pl.ds(col, B_PER_W)], idx_v)

    def idx_chunk(j):
        return idx_v.at[0, pl.ds(j * CHUNK, CHUNK)]

    cur = pltpu.async_copy(table_hbm.at[idx_chunk(0)], rows_v.at[0], sem)
    for j in range(NCHUNK):
        cur.wait()
        if j + 1 < NCHUNK:
            cur = pltpu.async_copy(
                table_hbm.at[idx_chunk(j + 1)], rows_v.at[(j + 1) % 2], sem)
        pltpu.sync_copy(
            rows_v.at[j % 2],
            out_hbm.at[pl.ds(wid * B_PER_W + j * CHUNK, CHUNK)],
        )


_gather_call = functools.partial(
    pl.kernel,
    out_type=jax.ShapeDtypeStruct((NT, DP), jnp.float32),
    mesh=plsc.VectorSubcoreMesh(core_axis_name="c", subcore_axis_name="s"),
    scratch_types=[
        pltpu.VMEM((1, B_PER_W), jnp.int32),
        pltpu.VMEM((2, CHUNK, DP), jnp.float32),
        pltpu.SemaphoreType.DMA,
    ],
)(_gather_body)


BT = 512
NBT = NT // BT


def _tpose0_body(g_ref, out_ref):
    gt = jnp.swapaxes(g_ref[...], 0, 1)
    out_ref[...] = lax.slice(gt, (0, 0), (D, BT))


def _tpose_body(g_ref, acc_ref, out_ref):
    del acc_ref
    gt = jnp.swapaxes(g_ref[...], 0, 1)
    out_ref[...] = lax.slice(gt, (0, 0), (D, BT))


_tpose_first = pl.pallas_call(
    _tpose0_body,
    grid=(NBT,),
    in_specs=[pl.BlockSpec((BT, DP), lambda i: (i, 0))],
    out_specs=pl.BlockSpec((D, BT), lambda i: (0, i)),
    out_shape=jax.ShapeDtypeStruct((D, N), jnp.float32),
)


def _make_tpose_call(c):
    return pl.pallas_call(
        _tpose_body,
        grid=(NBT,),
        in_specs=[
            pl.BlockSpec((BT, DP), lambda i: (i, 0)),
            pl.BlockSpec(memory_space=pl.ANY),
        ],
        out_specs=pl.BlockSpec((D, BT), lambda i, c=c: (0, c * NBT + i)),
        out_shape=jax.ShapeDtypeStruct((D, N), jnp.float32),
        input_output_aliases={1: 0},
    )


_tpose_calls = [_make_tpose_call(c) for c in range(1, NCHUNKS)]


def kernel(z_e_x, codebook):
    zt = z_e_x.T
    cbt = codebook.T
    cb_pad = jnp.pad(codebook, ((0, 0), (0, DP - D)))
    zq_t = None
    for c in range(NCHUNKS):
        idx_c = _argmin_calls[c](zt, cbt)
        g_c = _gather_call(cb_pad, idx_c)
        if c == 0:
            zq_t = _tpose_first(g_c)
        else:
            zq_t = _tpose_calls[c - 1](g_c, zq_t)
    z_q_x = zq_t.T
    return (z_q_x, z_q_x)


# speedup vs baseline: 1.1036x; 1.1036x over previous
import functools

import jax
import jax.numpy as jnp
from jax import lax
from jax.experimental import pallas as pl
from jax.experimental.pallas import tpu as pltpu
from jax.experimental.pallas import tpu_sc as plsc

N = 32768
D = 64
K = 1024

NCHUNKS = 4
NT = N // NCHUNKS

BN = 512
NB = NT // BN

NC = 2
NS = 16
NW = NC * NS
B_PER_W = NT // NW
CHUNK = 128
NCHUNK = B_PER_W // CHUNK
W_PER_ROW = BN // B_PER_W
DP = 128


def _argmin_body(zt_ref, cbt_ref, out_ref):
    zt = zt_ref[...]
    zsq = jnp.sum(zt * zt, axis=0, keepdims=True)
    cbt = cbt_ref[...]
    ones = jnp.ones((D, 1), jnp.float32)
    cbsq = lax.dot_general(
        cbt * cbt, ones, (((0,), (0,)), ((), ())),
        preferred_element_type=jnp.float32)
    mmneg2 = lax.dot_general(
        cbt * -2.0, zt, (((0,), (0,)), ((), ())),
        preferred_element_type=jnp.float32)
    zsq8 = jnp.broadcast_to(zsq, (8, BN))
    iota8 = lax.broadcasted_iota(jnp.int32, (8, BN), 0)
    rm = jnp.full((8, BN), jnp.inf, jnp.float32)
    ri = jnp.full((8, BN), K, jnp.int32)
    for j in range(K // 8):
        mm_j = lax.slice(mmneg2, (j * 8, 0), (j * 8 + 8, BN))
        cb_j = lax.slice(cbsq, (j * 8, 0), (j * 8 + 8, 1))
        d = (zsq8 + mm_j) + cb_j
        lt = d < rm
        rm = jnp.where(lt, d, rm)
        ri = jnp.where(lt, iota8 + (j * 8), ri)
    mn = jnp.min(rm, axis=0, keepdims=True)
    idx = jnp.min(jnp.where(rm == mn, ri, K), axis=0, keepdims=True)
    out_ref[pl.ds(pl.program_id(0), 1), :] = idx


def _make_argmin_call(c):
    return pl.pallas_call(
        _argmin_body,
        grid=(NB,),
        in_specs=[
            pl.BlockSpec((D, BN), lambda i, c=c: (0, c * NB + i)),
            pl.BlockSpec((D, K), lambda i: (0, 0)),
        ],
        out_specs=pl.BlockSpec((NB, BN), lambda i: (0, 0)),
        out_shape=jax.ShapeDtypeStruct((NB, BN), jnp.int32),
    )


_argmin_calls = [_make_argmin_call(c) for c in range(NCHUNKS)]


def _gather_body(table_hbm, idx_hbm, out_hbm, idx_v, rows_v, sem):
    wid = lax.axis_index("s") * NC + lax.axis_index("c")
    row = wid // W_PER_ROW
    col = (wid % W_PER_ROW) * B_PER_W
    pltpu.sync_copy(
        idx_hbm.at[pl.ds(row, 1), pl.ds(col, B_PER_W)], idx_v)

    def idx_chunk(j):
        return idx_v.at[0, pl.ds(j * CHUNK, CHUNK)]

    cur = pltpu.async_copy(table_hbm.at[idx_chunk(0)], rows_v.at[0], sem)
    for j in range(NCHUNK):
        cur.wait()
        if j + 1 < NCHUNK:
            cur = pltpu.async_copy(
                table_hbm.at[idx_chunk(j + 1)], rows_v.at[(j + 1) % 2], sem)
        pltpu.sync_copy(
            rows_v.at[j % 2],
            out_hbm.at[pl.ds(wid * B_PER_W + j * CHUNK, CHUNK)],
        )


_gather_call = functools.partial(
    pl.kernel,
    out_type=jax.ShapeDtypeStruct((NT, DP), jnp.float32),
    mesh=plsc.VectorSubcoreMesh(core_axis_name="c", subcore_axis_name="s"),
    scratch_types=[
        pltpu.VMEM((1, B_PER_W), jnp.int32),
        pltpu.VMEM((2, CHUNK, DP), jnp.float32),
        pltpu.SemaphoreType.DMA,
    ],
)(_gather_body)


def kernel(z_e_x, codebook):
    zt = z_e_x.T
    cbt = codebook.T
    cb_pad = jnp.pad(codebook, ((0, 0), (0, DP - D)))
    parts = []
    for c in range(NCHUNKS):
        idx_c = _argmin_calls[c](zt, cbt)
        parts.append(_gather_call(cb_pad, idx_c)[:, :D])
    z_q_x = jnp.concatenate(parts, axis=0)
    return (z_q_x, z_q_x)
